# baseline (speedup 1.0000x reference)
import functools

import jax
import jax.numpy as jnp
from jax import lax
from jax.experimental import pallas as pl
from jax.experimental.pallas import tpu as pltpu
from jax.experimental.pallas import tpu_sc as plsc

_L = 16
_NS = 16


def _sc_body(p_hbm, t_hbm, out_hbm, bufp, buft, stage, partial, fin, outv,
             *, rows_per_w, n, inv_n, inv_b):
    c = lax.axis_index("c")
    s = lax.axis_index("s")
    chunk = rows_per_w * n
    lane = lax.iota(jnp.int32, _L)

    def hsum(vec):
        tot = vec[0]
        for l in range(1, _L):
            tot = tot + vec[l]
        return tot

    @pl.when(c == 0)
    def _work():
        base = s * chunk
        pltpu.sync_copy(p_hbm.at[pl.ds(base, chunk)], bufp)
        pltpu.sync_copy(t_hbm.at[pl.ds(base, chunk)], buft)
        qv = jnp.zeros((_L,), jnp.float32)
        v = jnp.float32(0.0)
        for r in range(rows_per_w):
            asp = jnp.zeros((_L,), jnp.float32)
            aqp = jnp.zeros((_L,), jnp.float32)
            ast = jnp.zeros((_L,), jnp.float32)
            aqt = jnp.zeros((_L,), jnp.float32)
            for i in range(n // _L):
                vp = bufp[pl.ds(r * n + i * _L, _L)]
                vt = buft[pl.ds(r * n + i * _L, _L)]
                asp = asp + vp
                aqp = aqp + vp * vp
                ast = ast + vt
                aqt = aqt + vt * vt
            qv = qv + aqp + aqt
            v = v + hsum(asp) * hsum(ast)
        u = hsum(qv)
        res = jnp.where(lane == 0, jnp.full((_L,), u, jnp.float32),
                        jnp.where(lane == 1, jnp.full((_L,), v, jnp.float32),
                                  jnp.zeros((_L,), jnp.float32)))
        stage[...] = res
        pltpu.sync_copy(stage, partial.at[pl.ds(s * _L, _L)])

    plsc.subcore_barrier()

    @pl.when((c == 0) & (s == 0))
    def _finish():
        pltpu.sync_copy(partial, fin)
        acc = jnp.zeros((_L,), jnp.float32)
        for j in range(_NS):
            acc = acc + fin[pl.ds(j * _L, _L)]
        final = (acc[0] - 2.0 * inv_n * acc[1]) * inv_b
        outv[...] = jnp.full((_L,), final, jnp.float32)
        pltpu.sync_copy(outv, out_hbm)


def kernel(pred, target):
    b, n, c = pred.shape
    rows = b * c
    rows_per_w = rows // _NS
    p = pred.transpose(0, 2, 1).reshape(rows * n)
    t = target.transpose(0, 2, 1).reshape(rows * n)
    mesh = plsc.VectorSubcoreMesh(core_axis_name="c", subcore_axis_name="s",
                                  num_cores=2, num_subcores=_NS)
    body = functools.partial(_sc_body, rows_per_w=rows_per_w, n=n,
                             inv_n=1.0 / n, inv_b=1.0 / b)
    out = pl.kernel(
        body,
        out_type=jax.ShapeDtypeStruct((_L,), jnp.float32),
        mesh=mesh,
        scratch_types=[
            pltpu.VMEM((rows_per_w * n,), jnp.float32),
            pltpu.VMEM((rows_per_w * n,), jnp.float32),
            pltpu.VMEM((_L,), jnp.float32),
            pltpu.VMEM_SHARED((_NS * _L,), jnp.float32),
            pltpu.VMEM((_NS * _L,), jnp.float32),
            pltpu.VMEM((_L,), jnp.float32),
        ],
    )(p, t)
    return out[0]


# speedup vs baseline: 4.7005x; 4.7005x over previous
import functools

import jax
import jax.numpy as jnp
from jax.experimental import pallas as pl


def _emd_reduce_kernel(p_ref, t_ref, o_ref, *, inv_n, inv_b):
    p = p_ref[:]
    t = t_ref[:]
    total = jnp.sum(p * p + t * t, keepdims=True)
    sp = jnp.sum(p, axis=1, keepdims=True)
    st = jnp.sum(t, axis=1, keepdims=True)
    cross = jnp.sum(sp * st, keepdims=True)
    o_ref[:, :] = (total - 2.0 * inv_n * cross) * inv_b


def kernel(pred, target):
    b, n, c = pred.shape
    p = pred.transpose(0, 2, 1).reshape(b * c, n)
    t = target.transpose(0, 2, 1).reshape(b * c, n)
    out = pl.pallas_call(
        functools.partial(_emd_reduce_kernel, inv_n=1.0 / n, inv_b=1.0 / b),
        out_shape=jax.ShapeDtypeStruct((1, 1), jnp.float32),
    )(p, t)
    return out[0, 0]
